# baseline (speedup 1.0000x reference)
import functools

import jax
import jax.numpy as jnp
from jax import lax
from jax.experimental import pallas as pl
from jax.experimental.pallas import tpu as pltpu
from jax.experimental.pallas import tpu_sc as plsc

N_DIM = 64

NUM_CORES = 2
NUM_SUBCORES = 16
NUM_WORKERS = NUM_CORES * NUM_SUBCORES


def _build_sc_gather(n_rows, vocab, d, chunk):
  assert n_rows % NUM_WORKERS == 0
  rows_per_w = n_rows // NUM_WORKERS
  assert rows_per_w % chunk == 0
  n_chunks = rows_per_w // chunk
  assert n_chunks % 2 == 0 and chunk % 8 == 0

  mesh = plsc.VectorSubcoreMesh(
      core_axis_name="c", subcore_axis_name="s",
      num_cores=NUM_CORES, num_subcores=NUM_SUBCORES)

  @functools.partial(
      pl.kernel,
      out_type=jax.ShapeDtypeStruct((n_rows, d), jnp.float32),
      mesh=mesh,
      compiler_params=pltpu.CompilerParams(use_tc_tiling_on_sc=False),
      scratch_types=[
          pltpu.VMEM((rows_per_w,), jnp.int32),
          pltpu.VMEM((chunk, d), jnp.float32),
          pltpu.VMEM((chunk, d), jnp.float32),
          pltpu.SemaphoreType.DMA,
          pltpu.SemaphoreType.DMA,
      ],
  )
  def gather_kernel(idx_hbm, table_hbm, out_hbm,
                    idx_v, rows0, rows1, gsem0, gsem1):
    wid = lax.axis_index("s") * NUM_CORES + lax.axis_index("c")
    base = wid * rows_per_w
    pltpu.sync_copy(idx_hbm.at[pl.ds(base, rows_per_w)], idx_v)

    bufs = (rows0, rows1)
    sems = (gsem0, gsem1)

    def start_gather(c, b):
      pltpu.async_copy(
          table_hbm.at[idx_v.at[pl.ds(c * chunk, chunk)]], bufs[b], sems[b])

    def wait_gather(b):
      pltpu.make_async_copy(
          table_hbm.at[pl.ds(0, chunk)], bufs[b], sems[b]).wait()

    start_gather(0, 0)

    @pl.loop(0, n_chunks, step=2)
    def _(c):
      start_gather(c + 1, 1)
      wait_gather(0)
      pltpu.sync_copy(rows0, out_hbm.at[pl.ds(base + c * chunk, chunk)])

      @pl.when(c + 2 < n_chunks)
      def _():
        start_gather(c + 2, 0)

      wait_gather(1)
      pltpu.sync_copy(rows1, out_hbm.at[pl.ds(base + (c + 1) * chunk, chunk)])

  return gather_kernel


def kernel(word_indices, table):
  b0, s = word_indices.shape
  vocab, d = table.shape
  n_rows = b0 * s
  idx_flat = word_indices.reshape(n_rows).astype(jnp.int32)
  gather = _build_sc_gather(n_rows, vocab, d, chunk=800)
  out = gather(idx_flat, table)
  return out.reshape(b0, s, d)
